# baseline (speedup 1.0000x reference)
import functools

import jax
import jax.numpy as jnp
from jax import lax
from jax.experimental import pallas as pl
from jax.experimental.pallas import tpu as pltpu
from jax.experimental.pallas import tpu_sc as plsc

NC = 2
NS = 16
NW = NC * NS
LANES = 16

SUB = 782
HALF = SUB * 128 // 2
TB_PER_W = 4
LT_CHUNK = 5


def _tc_dtable(w_ref, p_ref):
    w0 = w_ref[0]
    w1 = w_ref[1]
    d = 1.0 / (1.0 + jnp.exp(w1 - w0))
    bits = jax.lax.bitcast_convert_type(d, jnp.uint32) + jnp.uint32(0x8000)
    lo = (bits[: SUB // 2] >> 16) & jnp.uint32(0xFFFF)
    hi = bits[SUB // 2 :] & jnp.uint32(0xFFFF0000)
    p_ref[...] = jax.lax.bitcast_convert_type(lo | hi, jnp.int32)


def _make_sc_gather(nlt, ntb):
    n_chunks = nlt // LT_CHUNK
    l_chunk = 8 * LT_CHUNK
    mesh = plsc.VectorSubcoreMesh(core_axis_name="c", subcore_axis_name="s")

    @functools.partial(
        pl.kernel,
        out_type=jax.ShapeDtypeStruct((8 * nlt, 2 * ntb, 128), jnp.float32),
        mesh=mesh,
        compiler_params=pltpu.CompilerParams(needs_layout_passes=False),
        scratch_types=[
            pltpu.VMEM((HALF,), jnp.int32),
            pltpu.VMEM((2, TB_PER_W, 8, 128), jnp.int32),
            pltpu.VMEM((2, 8, 2 * TB_PER_W, 128), jnp.float32),
            pltpu.SemaphoreType.DMA,
            pltpu.SemaphoreType.DMA,
            pltpu.SemaphoreType.DMA,
        ],
    )
    def sc_gather(tbl_hbm, idx_hbm, out_hbm, tbl_v, idx_v, out_v,
                  in_sem, out_sem, tab_sem):
        wid = lax.axis_index("s") * NC + lax.axis_index("c")
        tb0 = wid * TB_PER_W
        n_units = nlt

        def unit_slices(u):
            src = idx_hbm.at[u, pl.ds(tb0, TB_PER_W), :, :]
            dst = out_hbm.at[pl.ds(u * 8, 8), pl.ds(2 * tb0, 2 * TB_PER_W), :]
            return src, dst

        def start_in(u, buf):
            src, _ = unit_slices(u)
            return pltpu.async_copy(src, idx_v.at[buf], in_sem)

        tab_handle = pltpu.async_copy(tbl_hbm, tbl_v, tab_sem)
        start_in(0, 0)
        start_in(1, 1)
        tab_handle.wait()

        half = jnp.int32(HALF)
        mask_hi = jnp.int32(-65536)

        def run_unit(u, b):
            static = isinstance(u, int)
            src, dst = unit_slices(u)
            pltpu.make_async_copy(src, idx_v.at[b], in_sem).wait()

            def drain_out():
                pltpu.make_async_copy(out_v.at[b], dst, out_sem).wait()

            if static:
                drain_out()
            else:
                pl.when(u >= 2)(drain_out)

            @plsc.parallel_loop(0, 8)
            def body(ls):
                for tbh in range(TB_PER_W):
                    for j in range(128 // LANES):
                        v = idx_v[b, tbh, ls, pl.ds(j * LANES, LANES)]
                        c = v >= half
                        vv = jnp.where(c, v - half, v)
                        g = plsc.load_gather(tbl_v, [vv])
                        bits = jnp.where(c, g & mask_hi, g << 16)
                        dv = plsc.bitcast(bits, jnp.float32)
                        out_v[b, ls, 2 * tbh, pl.ds(j * LANES, LANES)] = dv
                        out_v[b, ls, 2 * tbh + 1, pl.ds(j * LANES, LANES)] = (
                            1.0 - dv
                        )

            pltpu.async_copy(out_v.at[b], dst, out_sem)

            def start_next():
                start_in(u + 2, b)

            if static:
                if u + 2 < n_units:
                    start_next()
            else:
                pl.when(u + 2 < n_units)(start_next)

        def group(g, carry):
            run_unit(2 * g, 0)
            run_unit(2 * g + 1, 1)
            return carry

        lax.fori_loop(0, n_units // 2, group, 0)
        if n_units % 2:
            run_unit(n_units - 1, 0)
        _, dst_a = unit_slices(n_units - 2)
        _, dst_b = unit_slices(n_units - 1)
        pltpu.make_async_copy(out_v.at[n_units % 2], dst_a, out_sem).wait()
        pltpu.make_async_copy(out_v.at[1 - n_units % 2], dst_b, out_sem).wait()

    return sc_gather


def kernel(feat_idx, W):
    b, h = feat_idx.shape
    v = W.shape[0]
    pad_v = SUB * 128
    ntb = b // 128
    nlt = h // 8
    assert v <= pad_v and ntb == NW * TB_PER_W and nlt % LT_CHUNK == 0

    w_t = jnp.pad(W, ((0, pad_v - v), (0, 0))).T.reshape(2, SUB, 128)
    tbl = pl.pallas_call(
        _tc_dtable,
        out_shape=jax.ShapeDtypeStruct((SUB // 2, 128), jnp.int32),
    )(w_t)

    idx4 = feat_idx.T.reshape(nlt, 8, ntb, 128).transpose(0, 2, 1, 3)

    out3 = _make_sc_gather(nlt, ntb)(tbl.reshape(HALF), idx4)
    return (
        out3.reshape(h, ntb, 2, 128)
        .transpose(1, 3, 0, 2)
        .reshape(b, h, 2)
    )


# speedup vs baseline: 1.8044x; 1.8044x over previous
import functools

import jax
import jax.numpy as jnp
from jax import lax
from jax.experimental import pallas as pl
from jax.experimental.pallas import tpu as pltpu
from jax.experimental.pallas import tpu_sc as plsc

NC = 2
NS = 16
NW = NC * NS
LANES = 16

SUB = 782
HALF = SUB * 128 // 2
TB_PER_W = 4
LT_CHUNK = 5


def _tc_dtable(w_ref, p_ref):
    w0 = w_ref[0]
    w1 = w_ref[1]
    d = 1.0 / (1.0 + jnp.exp(w1 - w0))
    bits = jax.lax.bitcast_convert_type(d, jnp.uint32) + jnp.uint32(0x8000)
    lo = (bits[: SUB // 2] >> 16) & jnp.uint32(0xFFFF)
    hi = bits[SUB // 2 :] & jnp.uint32(0xFFFF0000)
    p_ref[...] = jax.lax.bitcast_convert_type(lo | hi, jnp.int32)


def _make_sc_gather(nlt, ntb):
    n_chunks = nlt // LT_CHUNK
    l_chunk = 8 * LT_CHUNK
    mesh = plsc.VectorSubcoreMesh(core_axis_name="c", subcore_axis_name="s")

    @functools.partial(
        pl.kernel,
        out_type=jax.ShapeDtypeStruct((8 * nlt, 2 * ntb, 128), jnp.float32),
        mesh=mesh,
        compiler_params=pltpu.CompilerParams(needs_layout_passes=False),
        scratch_types=[
            pltpu.VMEM((HALF,), jnp.int32),
            pltpu.VMEM((2, LT_CHUNK, 2, 8, 128), jnp.int32),
            pltpu.VMEM((2, l_chunk, 4, 128), jnp.float32),
            pltpu.VMEM_SHARED((HALF,), jnp.int32),
            pltpu.SemaphoreType.DMA,
            pltpu.SemaphoreType.DMA,
            pltpu.SemaphoreType.DMA,
        ],
    )
    def sc_gather(tbl_hbm, idx_hbm, out_hbm, tbl_v, idx_v, out_v, tbl_sp,
                  in_sem, out_sem, tab_sem):
        sid = lax.axis_index("s")
        wid = sid * NC + lax.axis_index("c")
        n_units = (TB_PER_W // 2) * n_chunks

        def unit_slices(u):
            pr = u // n_chunks
            lc = u - pr * n_chunks
            tb = wid * TB_PER_W + 2 * pr
            src = idx_hbm.at[
                pl.ds(lc * LT_CHUNK, LT_CHUNK), pl.ds(tb, 2), :, :
            ]
            dst = out_hbm.at[
                pl.ds(lc * l_chunk, l_chunk), pl.ds(2 * tb, 4), :
            ]
            return src, dst

        def start_in(u, buf):
            src, _ = unit_slices(u)
            return pltpu.async_copy(src, idx_v.at[buf], in_sem)

        start_in(0, 0)
        start_in(1, 1)

        tsl = HALF // NS
        toff = sid * tsl
        pltpu.sync_copy(tbl_hbm.at[pl.ds(toff, tsl)], tbl_v.at[pl.ds(toff, tsl)])
        pltpu.sync_copy(tbl_v.at[pl.ds(toff, tsl)], tbl_sp.at[pl.ds(toff, tsl)])
        plsc.subcore_barrier()
        pltpu.async_copy(tbl_sp, tbl_v, tab_sem).wait()

        half = jnp.int32(HALF)
        mask_hi = jnp.int32(-65536)

        def run_unit(u, b):
            src, dst = unit_slices(u)
            pltpu.make_async_copy(src, idx_v.at[b], in_sem).wait()

            @pl.when(u >= 2)
            def _():
                pltpu.make_async_copy(out_v.at[b], dst, out_sem).wait()

            @plsc.parallel_loop(0, l_chunk)
            def body(ll):
                lt = ll >> 3
                ls = ll & 7
                for tbh in range(2):
                    for j in range(128 // LANES):
                        v = idx_v[b, lt, tbh, ls, pl.ds(j * LANES, LANES)]
                        c = v >= half
                        vv = jnp.where(c, v - half, v)
                        g = plsc.load_gather(tbl_v, [vv])
                        bits = jnp.where(c, g & mask_hi, g << 16)
                        dv = plsc.bitcast(bits, jnp.float32)
                        out_v[b, ll, 2 * tbh, pl.ds(j * LANES, LANES)] = dv
                        out_v[b, ll, 2 * tbh + 1, pl.ds(j * LANES, LANES)] = (
                            1.0 - dv
                        )

            pltpu.async_copy(out_v.at[b], dst, out_sem)

            @pl.when(u + 2 < n_units)
            def _():
                start_in(u + 2, b)

        def group(g, carry):
            run_unit(2 * g, 0)
            run_unit(2 * g + 1, 1)
            return carry

        lax.fori_loop(0, n_units // 2, group, 0)
        _, dst0 = unit_slices(n_units - 2)
        _, dst1 = unit_slices(n_units - 1)
        pltpu.make_async_copy(out_v.at[0], dst0, out_sem).wait()
        pltpu.make_async_copy(out_v.at[1], dst1, out_sem).wait()

    return sc_gather


def kernel(feat_idx, W):
    b, h = feat_idx.shape
    v = W.shape[0]
    pad_v = SUB * 128
    ntb = b // 128
    nlt = h // 8
    assert v <= pad_v and ntb == NW * TB_PER_W and nlt % LT_CHUNK == 0

    w_t = jnp.pad(W, ((0, pad_v - v), (0, 0))).T.reshape(2, SUB, 128)
    tbl = pl.pallas_call(
        _tc_dtable,
        out_shape=jax.ShapeDtypeStruct((SUB // 2, 128), jnp.int32),
    )(w_t)

    idx4 = feat_idx.T.reshape(nlt, 8, ntb, 128).transpose(0, 2, 1, 3)

    out3 = _make_sc_gather(nlt, ntb)(tbl.reshape(HALF), idx4)
    return (
        out3.reshape(h, ntb, 2, 128)
        .transpose(1, 3, 0, 2)
        .reshape(b, h, 2)
    )
